# baseline (speedup 1.0000x reference)
import functools

import jax
import jax.numpy as jnp
from jax import lax
from jax.experimental import pallas as pl
from jax.experimental.pallas import tpu as pltpu
from jax.experimental.pallas import tpu_sc as plsc

NC = 2
NS = 16
L = 16
NW = NC * NS

B = 16384
D = 32
NT = 1000000
BPW = B // NW
GPW = BPW // L
K = 2 * D

_mesh = plsc.VectorSubcoreMesh(core_axis_name="c", subcore_axis_name="s")


@functools.partial(
    pl.kernel,
    out_type=jax.ShapeDtypeStruct((B,), jnp.float32),
    mesh=_mesh,
    scratch_types=(
        [pltpu.VMEM((BPW,), jnp.int32)] * 2
        + [pltpu.VMEM((BPW,), jnp.int32)] * K
        + [pltpu.VMEM((BPW,), jnp.float32)] * K
        + [
            pltpu.VMEM((K * L,), jnp.float32),
            pltpu.VMEM((L,), jnp.float32),
            pltpu.VMEM((BPW,), jnp.float32),
            pltpu.SemaphoreType.DMA,
            pltpu.SemaphoreType.DMA,
        ]
    ),
    compiler_params=pltpu.CompilerParams(needs_layout_passes=False),
)
def _ncf_sc(uids, iids, utab, itab, wb, bb, out, *scratch):
    uidx_v, iidx_v = scratch[0], scratch[1]
    idx_refs = scratch[2:2 + K]
    col_refs = scratch[2 + K:2 + 2 * K]
    w_v, b_v, out_v, sem_u, sem_i = scratch[2 + 2 * K:]
    wid = lax.axis_index("s") * NC + lax.axis_index("c")
    base = wid * BPW

    pltpu.sync_copy(uids.at[pl.ds(base, BPW)], uidx_v)
    pltpu.sync_copy(iids.at[pl.ds(base, BPW)], iidx_v)
    pltpu.sync_copy(wb, w_v)
    pltpu.sync_copy(bb, b_v)

    def build(g, carry):
        sl = pl.ds(g * L, L)
        u = uidx_v[sl]
        i = iidx_v[sl]
        for d in range(D):
            idx_refs[d][sl] = u + d * NT
        for d in range(D):
            idx_refs[D + d][sl] = i + d * NT
        return carry

    lax.fori_loop(0, GPW, build, 0)

    copies = []
    for d in range(D):
        copies.append(pltpu.async_copy(
            utab.at[idx_refs[d]], col_refs[d], sem_u))
    for d in range(D):
        copies.append(pltpu.async_copy(
            itab.at[idx_refs[D + d]], col_refs[D + d], sem_i))
    for c in copies:
        c.wait()

    def group(g, carry):
        sl = pl.ds(g * L, L)
        acc = b_v[...]
        for k in range(K):
            acc = acc + col_refs[k][sl] * w_v[pl.ds(k * L, L)]
        out_v[sl] = acc
        return carry

    lax.fori_loop(0, GPW, group, 0)

    pltpu.sync_copy(out_v, out.at[pl.ds(base, BPW)])


def kernel(user_ids, item_ids, user_table, item_table, W, b):
    ut = user_table.T.reshape(-1)
    it = item_table.T.reshape(-1)
    wb = jnp.broadcast_to(W.reshape(K, 1), (K, L)).reshape(-1)
    bb = jnp.broadcast_to(b, (L,))
    out = _ncf_sc(user_ids, item_ids, ut, it, wb, bb)
    return out.reshape(B, 1)


# speedup vs baseline: 45.9584x; 45.9584x over previous
import functools

import jax
import jax.numpy as jnp
from jax import lax
from jax.experimental import pallas as pl
from jax.experimental.pallas import tpu as pltpu
from jax.experimental.pallas import tpu_sc as plsc

NC = 2
NS = 16
L = 16
NW = NC * NS

B = 16384
D = 32
NT = 1000000
BPW = B // NW
GPW = BPW // L
BN = 16384

_mesh = plsc.VectorSubcoreMesh(core_axis_name="c", subcore_axis_name="s")


def _sweep_body(ut_ref, it_ref, wu_ref, wi_ref, su_ref, si_ref):
    su_ref[...] = jnp.dot(wu_ref[...], ut_ref[...],
                          preferred_element_type=jnp.float32)[0]
    si_ref[...] = jnp.dot(wi_ref[...], it_ref[...],
                          preferred_element_type=jnp.float32)[0]


_sweep = pl.pallas_call(
    _sweep_body,
    grid=(pl.cdiv(NT, BN),),
    in_specs=[
        pl.BlockSpec((D, BN), lambda i: (0, i)),
        pl.BlockSpec((D, BN), lambda i: (0, i)),
        pl.BlockSpec((1, D), lambda i: (0, 0)),
        pl.BlockSpec((1, D), lambda i: (0, 0)),
    ],
    out_specs=[
        pl.BlockSpec((BN,), lambda i: (i,)),
        pl.BlockSpec((BN,), lambda i: (i,)),
    ],
    out_shape=[jax.ShapeDtypeStruct((NT,), jnp.float32)] * 2,
)


@functools.partial(
    pl.kernel,
    out_type=jax.ShapeDtypeStruct((B,), jnp.float32),
    mesh=_mesh,
    scratch_types=[
        pltpu.VMEM((BPW,), jnp.int32),
        pltpu.VMEM((BPW,), jnp.int32),
        pltpu.VMEM((BPW,), jnp.float32),
        pltpu.VMEM((BPW,), jnp.float32),
        pltpu.VMEM((L,), jnp.float32),
        pltpu.VMEM((BPW,), jnp.float32),
        pltpu.SemaphoreType.DMA,
        pltpu.SemaphoreType.DMA,
    ],
    compiler_params=pltpu.CompilerParams(needs_layout_passes=False),
)
def _gather_sc(uids, iids, su, si, bb, out,
               uidx_v, iidx_v, sug_v, sig_v, b_v, out_v, sem_u, sem_i):
    wid = lax.axis_index("s") * NC + lax.axis_index("c")
    base = wid * BPW

    pltpu.sync_copy(uids.at[pl.ds(base, BPW)], uidx_v)
    pltpu.sync_copy(iids.at[pl.ds(base, BPW)], iidx_v)
    cu = pltpu.async_copy(su.at[uidx_v], sug_v, sem_u)
    ci = pltpu.async_copy(si.at[iidx_v], sig_v, sem_i)
    pltpu.sync_copy(bb, b_v)
    cu.wait()
    ci.wait()

    def group(g, carry):
        sl = pl.ds(g * L, L)
        out_v[sl] = sug_v[sl] + sig_v[sl] + b_v[...]
        return carry

    lax.fori_loop(0, GPW, group, 0)

    pltpu.sync_copy(out_v, out.at[pl.ds(base, BPW)])


def kernel(user_ids, item_ids, user_table, item_table, W, b):
    ut = user_table.T
    it = item_table.T
    wu = W[:, :D]
    wi = W[:, D:]
    su, si = _sweep(ut, it, wu, wi)
    bb = jnp.broadcast_to(b, (L,))
    out = _gather_sc(user_ids, item_ids, su, si, bb)
    return out.reshape(B, 1)


# speedup vs baseline: 50.2204x; 1.0927x over previous
import functools

import jax
import jax.numpy as jnp
from jax import lax
from jax.experimental import pallas as pl
from jax.experimental.pallas import tpu as pltpu
from jax.experimental.pallas import tpu_sc as plsc

NC = 2
NS = 16
L = 16
NW = NC * NS

B = 16384
D = 32
NT = 1000000
BPW = B // NW
GPW = BPW // L
BN = 32768

_mesh = plsc.VectorSubcoreMesh(core_axis_name="c", subcore_axis_name="s")


def _sweep_body(ut_ref, it_ref, wu_ref, wi_ref, su_ref, si_ref):
    su_ref[...] = jnp.dot(wu_ref[...], ut_ref[...],
                          preferred_element_type=jnp.float32)[0]
    si_ref[...] = jnp.dot(wi_ref[...], it_ref[...],
                          preferred_element_type=jnp.float32)[0]


_sweep = pl.pallas_call(
    _sweep_body,
    grid=(pl.cdiv(NT, BN),),
    in_specs=[
        pl.BlockSpec((D, BN), lambda i: (0, i)),
        pl.BlockSpec((D, BN), lambda i: (0, i)),
        pl.BlockSpec((1, D), lambda i: (0, 0)),
        pl.BlockSpec((1, D), lambda i: (0, 0)),
    ],
    out_specs=[
        pl.BlockSpec((BN,), lambda i: (i,)),
        pl.BlockSpec((BN,), lambda i: (i,)),
    ],
    out_shape=[jax.ShapeDtypeStruct((NT,), jnp.float32)] * 2,
)


@functools.partial(
    pl.kernel,
    out_type=jax.ShapeDtypeStruct((B,), jnp.float32),
    mesh=_mesh,
    scratch_types=[
        pltpu.VMEM((BPW,), jnp.int32),
        pltpu.VMEM((BPW,), jnp.int32),
        pltpu.VMEM((BPW,), jnp.float32),
        pltpu.VMEM((BPW,), jnp.float32),
        pltpu.VMEM((L,), jnp.float32),
        pltpu.VMEM((BPW,), jnp.float32),
        pltpu.SemaphoreType.DMA,
        pltpu.SemaphoreType.DMA,
    ],
    compiler_params=pltpu.CompilerParams(needs_layout_passes=False),
)
def _gather_sc(uids, iids, su, si, bb, out,
               uidx_v, iidx_v, sug_v, sig_v, b_v, out_v, sem_u, sem_i):
    wid = lax.axis_index("s") * NC + lax.axis_index("c")
    base = wid * BPW

    pltpu.sync_copy(uids.at[pl.ds(base, BPW)], uidx_v)
    pltpu.sync_copy(iids.at[pl.ds(base, BPW)], iidx_v)
    cu = pltpu.async_copy(su.at[uidx_v], sug_v, sem_u)
    ci = pltpu.async_copy(si.at[iidx_v], sig_v, sem_i)
    pltpu.sync_copy(bb, b_v)
    cu.wait()
    ci.wait()

    def group(g, carry):
        sl = pl.ds(g * L, L)
        out_v[sl] = sug_v[sl] + sig_v[sl] + b_v[...]
        return carry

    lax.fori_loop(0, GPW, group, 0)

    pltpu.sync_copy(out_v, out.at[pl.ds(base, BPW)])


def kernel(user_ids, item_ids, user_table, item_table, W, b):
    ut = user_table.T
    it = item_table.T
    wu = W[:, :D]
    wi = W[:, D:]
    su, si = _sweep(ut, it, wu, wi)
    bb = jnp.broadcast_to(b, (L,))
    out = _gather_sc(user_ids, item_ids, su, si, bb)
    return out.reshape(B, 1)
